# baseline (device time: 172489 ns/iter reference)
import jax
import jax.numpy as jnp
from jax import lax
from jax.experimental import pallas as pl
from jax.experimental.pallas import tpu as pltpu

N_DEV = 4
N_LANES = 8

A_SLOT = (0, 1, 0)
B_SLOT = (1, 0, 1)

LANES = ((0, +1), (4, -1), (1, +1), (5, -1),
         (2, +1), (6, -1), (3, +1), (7, -1))


def kernel(A, B):
    m, k = A.shape
    _, n = B.shape
    m_blk = m // N_DEV
    w = n // N_LANES
    a_piece = 256

    def body(a_any, b_any, out_ref,
             a16, b16, stg_a, stg_b, comm,
             stg_a_sem, stg_b_sem, send_sems, recv_sems, out_sems):
        my = lax.axis_index("i")
        left = lax.rem(my + N_DEV - 1, N_DEV)
        right = lax.rem(my + 1, N_DEV)

        barrier_sem = pltpu.get_barrier_semaphore()
        for nbr in (left, right):
            pl.semaphore_signal(
                barrier_sem, inc=1,
                device_id=(nbr,), device_id_type=pl.DeviceIdType.MESH,
            )

        def stage_b_start(col):
            pltpu.make_async_copy(
                b_any.at[:, pl.ds(col * w, w)], stg_b, stg_b_sem
            ).start()

        def stage_b_finish(col):
            pltpu.make_async_copy(
                b_any.at[:, pl.ds(col * w, w)], stg_b, stg_b_sem
            ).wait()
            b16[:, pl.ds(col * w, w)] = stg_b[:, :].astype(jnp.bfloat16)

        def a_rows(blk, piece):
            return lax.rem(blk, N_DEV) * m_blk + piece * a_piece

        def a_piece_start(blk, piece):
            pltpu.make_async_copy(
                a_any.at[pl.ds(a_rows(blk, piece), a_piece), :],
                stg_a, stg_a_sem,
            ).start()

        def a_piece_finish(blk, piece):
            pltpu.make_async_copy(
                a_any.at[pl.ds(a_rows(blk, piece), a_piece), :],
                stg_a, stg_a_sem,
            ).wait()
            a16[pl.ds(a_rows(blk, piece), a_piece), :] = (
                stg_a[:, :].astype(jnp.bfloat16))

        def stage_a_block(blk):
            for p in range(m_blk // a_piece):
                a_piece_start(blk, p)
                a_piece_finish(blk, p)

        def lane_dot(c, col):
            return jnp.dot(
                a16[pl.ds(lax.rem(c, N_DEV) * m_blk, m_blk),
                    :],
                b16[:, col * w:(col + 1) * w],
                preferred_element_type=jnp.float32,
            )

        def hop_rdma(li, s):
            _, d = LANES[li]
            return pltpu.make_async_remote_copy(
                src_ref=comm.at[li, A_SLOT[s]],
                dst_ref=comm.at[li, B_SLOT[s]],
                send_sem=send_sems.at[li, s],
                recv_sem=recv_sems.at[li, s],
                device_id=(right,) if d > 0 else (left,),
                device_id_type=pl.DeviceIdType.MESH,
            )

        stage_b_start(LANES[0][0])
        stage_a_block(my + 3)
        stage_a_block(my + 1)
        pl.semaphore_wait(barrier_sem, 2)

        n_pieces = m_blk // a_piece
        a_seq = [(my + 2, p) for p in range(n_pieces)] + \
                [(my, p) for p in range(n_pieces)]

        rdmas = [None] * N_LANES
        for li in range(N_LANES):
            col, d = LANES[li]
            stage_b_finish(col)
            if li + 1 < N_LANES:
                stage_b_start(LANES[li + 1][0])
            if li >= 1:
                a_piece_finish(*a_seq[li - 1])
            a_piece_start(*a_seq[li])
            comm[li, A_SLOT[0]] = (
                lane_dot(my + 3 if d > 0 else my + 1, col)
                .astype(jnp.bfloat16))
            rdmas[li] = hop_rdma(li, 0)
            rdmas[li].start()
        a_piece_finish(*a_seq[N_LANES - 1])

        for s in range(N_DEV - 1):
            for li in range(N_LANES):
                col, d = LANES[li]
                c = my + 2 - s if d > 0 else my + 2 + s
                rdmas[li].wait_recv()
                comm[li, B_SLOT[s]] = (
                    lane_dot(c, col)
                    + comm[li, B_SLOT[s]].astype(jnp.float32)
                ).astype(jnp.bfloat16)
                rdmas[li].wait_send()
                if s < N_DEV - 2:
                    rdmas[li] = hop_rdma(li, s + 1)
                    rdmas[li].start()
                else:
                    pltpu.make_async_copy(
                        comm.at[li, B_SLOT[s]],
                        out_ref.at[:, pl.ds(col * w, w)],
                        out_sems.at[li],
                    ).start()

        for li in range(N_LANES):
            pltpu.make_async_copy(
                comm.at[li, B_SLOT[2]],
                out_ref.at[:, pl.ds(LANES[li][0] * w, w)],
                out_sems.at[li],
            ).wait()

    return pl.pallas_call(
        body,
        out_shape=jax.ShapeDtypeStruct((m_blk, n), jnp.bfloat16),
        in_specs=[
            pl.BlockSpec(memory_space=pl.ANY),
            pl.BlockSpec(memory_space=pl.ANY),
        ],
        out_specs=pl.BlockSpec(memory_space=pl.ANY),
        scratch_shapes=[
            pltpu.VMEM((m, k), jnp.bfloat16),
            pltpu.VMEM((k, n), jnp.bfloat16),
            pltpu.VMEM((256, k), jnp.float32),
            pltpu.VMEM((k, n // N_LANES), jnp.float32),
            pltpu.VMEM((N_LANES, 2, m_blk, n // N_LANES), jnp.bfloat16),
            pltpu.SemaphoreType.DMA,
            pltpu.SemaphoreType.DMA,
            pltpu.SemaphoreType.DMA((N_LANES, 3)),
            pltpu.SemaphoreType.DMA((N_LANES, 3)),
            pltpu.SemaphoreType.DMA((N_LANES,)),
        ],
        compiler_params=pltpu.CompilerParams(
            collective_id=0,
            vmem_limit_bytes=100 * 1024 * 1024,
        ),
    )(A, B)


# device time: 166873 ns/iter; 1.0337x vs baseline; 1.0337x over previous
import jax
import jax.numpy as jnp
from jax import lax
from jax.experimental import pallas as pl
from jax.experimental.pallas import tpu as pltpu

N_DEV = 4
N_LANES = 8

A_SLOT = (0, 1, 0)
B_SLOT = (1, 0, 1)

LANES = ((0, +1), (4, -1), (1, +1), (5, -1),
         (2, +1), (6, -1), (3, +1), (7, -1))


def kernel(A, B):
    m, k = A.shape
    _, n = B.shape
    m_blk = m // N_DEV
    w = n // N_LANES
    a_piece = 256

    def body(a_any, b_any, out_ref,
             a16, b16, stg_a, stg_b, comm,
             stg_a_sem, stg_b_sem, send_sems, recv_sems, out_sems):
        my = lax.axis_index("i")
        left = lax.rem(my + N_DEV - 1, N_DEV)
        right = lax.rem(my + 1, N_DEV)

        barrier_sem = pltpu.get_barrier_semaphore()
        for nbr in (left, right):
            pl.semaphore_signal(
                barrier_sem, inc=1,
                device_id=(nbr,), device_id_type=pl.DeviceIdType.MESH,
            )

        def stage_b_start(col):
            pltpu.make_async_copy(
                b_any.at[:, pl.ds(col * w, w)], stg_b, stg_b_sem
            ).start()

        def stage_b_finish(col):
            pltpu.make_async_copy(
                b_any.at[:, pl.ds(col * w, w)], stg_b, stg_b_sem
            ).wait()
            b16[:, pl.ds(col * w, w)] = stg_b[:, :].astype(jnp.bfloat16)

        def a_rows(blk, piece):
            return lax.rem(blk, N_DEV) * m_blk + piece * a_piece

        def a_piece_start(blk, piece, buf):
            pltpu.make_async_copy(
                a_any.at[pl.ds(a_rows(blk, piece), a_piece), :],
                stg_a.at[buf], stg_a_sem.at[buf],
            ).start()

        def a_piece_finish(blk, piece, buf):
            pltpu.make_async_copy(
                a_any.at[pl.ds(a_rows(blk, piece), a_piece), :],
                stg_a.at[buf], stg_a_sem.at[buf],
            ).wait()
            a16[pl.ds(a_rows(blk, piece), a_piece), :] = (
                stg_a[buf].astype(jnp.bfloat16))

        def lane_dot(c, col):
            return jnp.dot(
                a16[pl.ds(lax.rem(c, N_DEV) * m_blk, m_blk),
                    :],
                b16[:, col * w:(col + 1) * w],
                preferred_element_type=jnp.float32,
            )

        def hop_rdma(li, s):
            _, d = LANES[li]
            return pltpu.make_async_remote_copy(
                src_ref=comm.at[li, A_SLOT[s]],
                dst_ref=comm.at[li, B_SLOT[s]],
                send_sem=send_sems.at[li, s],
                recv_sem=recv_sems.at[li, s],
                device_id=(right,) if d > 0 else (left,),
                device_id_type=pl.DeviceIdType.MESH,
            )

        n_pieces = m_blk // a_piece
        pro_seq = [(my + 3, p) for p in range(n_pieces)] + \
                  [(my + 1, p) for p in range(n_pieces)]
        stage_b_start(LANES[0][0])
        for i, (blk, p) in enumerate(pro_seq):
            a_piece_start(blk, p, i % 2)
            if i >= 1:
                a_piece_finish(*pro_seq[i - 1], (i - 1) % 2)
        a_piece_finish(*pro_seq[-1], (len(pro_seq) - 1) % 2)
        pl.semaphore_wait(barrier_sem, 2)

        a_seq = [(my + 2, p) for p in range(n_pieces)] + \
                [(my, p) for p in range(n_pieces)]

        rdmas = [None] * N_LANES
        for li in range(N_LANES):
            col, d = LANES[li]
            stage_b_finish(col)
            if li + 1 < N_LANES:
                stage_b_start(LANES[li + 1][0])
            if li >= 1:
                a_piece_finish(*a_seq[li - 1], (li - 1) % 2)
            a_piece_start(*a_seq[li], li % 2)
            comm[li, A_SLOT[0]] = (
                lane_dot(my + 3 if d > 0 else my + 1, col)
                .astype(jnp.bfloat16))
            rdmas[li] = hop_rdma(li, 0)
            rdmas[li].start()
        a_piece_finish(*a_seq[N_LANES - 1], (N_LANES - 1) % 2)

        for s in range(N_DEV - 1):
            for li in range(N_LANES):
                col, d = LANES[li]
                c = my + 2 - s if d > 0 else my + 2 + s
                rdmas[li].wait_recv()
                comm[li, B_SLOT[s]] = (
                    lane_dot(c, col)
                    + comm[li, B_SLOT[s]].astype(jnp.float32)
                ).astype(jnp.bfloat16)
                rdmas[li].wait_send()
                if s < N_DEV - 2:
                    rdmas[li] = hop_rdma(li, s + 1)
                    rdmas[li].start()
                else:
                    pltpu.make_async_copy(
                        comm.at[li, B_SLOT[s]],
                        out_ref.at[:, pl.ds(col * w, w)],
                        out_sems.at[li],
                    ).start()

        for li in range(N_LANES):
            pltpu.make_async_copy(
                comm.at[li, B_SLOT[2]],
                out_ref.at[:, pl.ds(LANES[li][0] * w, w)],
                out_sems.at[li],
            ).wait()

    return pl.pallas_call(
        body,
        out_shape=jax.ShapeDtypeStruct((m_blk, n), jnp.bfloat16),
        in_specs=[
            pl.BlockSpec(memory_space=pl.ANY),
            pl.BlockSpec(memory_space=pl.ANY),
        ],
        out_specs=pl.BlockSpec(memory_space=pl.ANY),
        scratch_shapes=[
            pltpu.VMEM((m, k), jnp.bfloat16),
            pltpu.VMEM((k, n), jnp.bfloat16),
            pltpu.VMEM((2, 256, k), jnp.float32),
            pltpu.VMEM((k, n // N_LANES), jnp.float32),
            pltpu.VMEM((N_LANES, 2, m_blk, n // N_LANES), jnp.bfloat16),
            pltpu.SemaphoreType.DMA((2,)),
            pltpu.SemaphoreType.DMA,
            pltpu.SemaphoreType.DMA((N_LANES, 3)),
            pltpu.SemaphoreType.DMA((N_LANES, 3)),
            pltpu.SemaphoreType.DMA((N_LANES,)),
        ],
        compiler_params=pltpu.CompilerParams(
            collective_id=0,
            vmem_limit_bytes=100 * 1024 * 1024,
        ),
    )(A, B)


# device time: 166228 ns/iter; 1.0377x vs baseline; 1.0039x over previous
import jax
import jax.numpy as jnp
from jax import lax
from jax.experimental import pallas as pl
from jax.experimental.pallas import tpu as pltpu

N_DEV = 4
N_LANES = 8

A_SLOT = (0, 1, 0)
B_SLOT = (1, 0, 1)

LANES = ((0, +1), (4, -1), (1, +1), (5, -1),
         (2, +1), (6, -1), (3, +1), (7, -1))


def kernel(A, B):
    m, k = A.shape
    _, n = B.shape
    m_blk = m // N_DEV
    w = n // N_LANES
    a_piece = 256

    def body(a_any, b_any, out_ref,
             a16, b16, stg_a, stg_b, comm,
             stg_a_sem, stg_b_sem, send_sems, recv_sems, out_sems):
        my = lax.axis_index("i")
        left = lax.rem(my + N_DEV - 1, N_DEV)
        right = lax.rem(my + 1, N_DEV)

        barrier_sem = pltpu.get_barrier_semaphore()
        for nbr in (left, right):
            pl.semaphore_signal(
                barrier_sem, inc=1,
                device_id=(nbr,), device_id_type=pl.DeviceIdType.MESH,
            )

        def stage_b_start(col):
            pltpu.make_async_copy(
                b_any.at[:, pl.ds(col * w, w)], stg_b, stg_b_sem
            ).start()

        def stage_b_finish(col):
            pltpu.make_async_copy(
                b_any.at[:, pl.ds(col * w, w)], stg_b, stg_b_sem
            ).wait()
            b16[:, pl.ds(col * w, w)] = stg_b[:, :].astype(jnp.bfloat16)

        def a_rows(blk, piece):
            return lax.rem(blk, N_DEV) * m_blk + piece * a_piece

        def a_piece_start(blk, piece, buf):
            pltpu.make_async_copy(
                a_any.at[pl.ds(a_rows(blk, piece), a_piece), :],
                stg_a.at[buf], stg_a_sem.at[buf],
            ).start()

        def a_piece_finish(blk, piece, buf):
            pltpu.make_async_copy(
                a_any.at[pl.ds(a_rows(blk, piece), a_piece), :],
                stg_a.at[buf], stg_a_sem.at[buf],
            ).wait()
            a16[pl.ds(a_rows(blk, piece), a_piece), :] = (
                stg_a[buf].astype(jnp.bfloat16))

        def lane_dot(c, col):
            return jnp.dot(
                a16[pl.ds(lax.rem(c, N_DEV) * m_blk, m_blk),
                    :],
                b16[:, col * w:(col + 1) * w],
                preferred_element_type=jnp.float32,
            )

        def hop_rdma(li, s):
            _, d = LANES[li]
            return pltpu.make_async_remote_copy(
                src_ref=comm.at[li, A_SLOT[s]],
                dst_ref=comm.at[li, B_SLOT[s]],
                send_sem=send_sems.at[li, s],
                recv_sem=recv_sems.at[li, s],
                device_id=(right,) if d > 0 else (left,),
                device_id_type=pl.DeviceIdType.MESH,
            )

        n_pieces = m_blk // a_piece
        pro_seq = [(my + 3, p) for p in range(n_pieces)] + \
                  [(my + 1, p) for p in range(n_pieces)]
        stage_b_start(LANES[0][0])
        for i, (blk, p) in enumerate(pro_seq):
            a_piece_start(blk, p, i % 2)
            if i >= 1:
                a_piece_finish(*pro_seq[i - 1], (i - 1) % 2)
        a_piece_finish(*pro_seq[-1], (len(pro_seq) - 1) % 2)
        pl.semaphore_wait(barrier_sem, 2)

        a_seq = [(my + 2, p) for p in range(n_pieces)] + \
                [(my, p) for p in range(n_pieces)]

        rdmas = [None] * N_LANES

        order = [("p", li) for li in range(4)]
        for li in range(4, 8):
            order += [("p", li), (0, li - 4)]
        for li in range(4, 8):
            order += [(0, li), (1, li - 4)]
        for li in range(4, 8):
            order += [(1, li), (2, li - 4)]
        order += [(2, li) for li in range(4, 8)]

        for ev, (kind, li) in enumerate(order):
            col, d = LANES[li]
            if ev < len(a_seq):
                a_piece_start(*a_seq[ev], ev % 2)
            if 1 <= ev <= len(a_seq):
                a_piece_finish(*a_seq[ev - 1], (ev - 1) % 2)

            if kind == "p":
                stage_b_finish(col)
                if li + 1 < N_LANES:
                    stage_b_start(LANES[li + 1][0])
                comm[li, A_SLOT[0]] = (
                    lane_dot(my + 3 if d > 0 else my + 1, col)
                    .astype(jnp.bfloat16))
                rdmas[li] = hop_rdma(li, 0)
                rdmas[li].start()
            else:
                s = kind
                c = my + 2 - s if d > 0 else my + 2 + s
                rdmas[li].wait_recv()
                comm[li, B_SLOT[s]] = (
                    lane_dot(c, col)
                    + comm[li, B_SLOT[s]].astype(jnp.float32)
                ).astype(jnp.bfloat16)
                rdmas[li].wait_send()
                if s < N_DEV - 2:
                    rdmas[li] = hop_rdma(li, s + 1)
                    rdmas[li].start()
                else:
                    pltpu.make_async_copy(
                        comm.at[li, B_SLOT[s]],
                        out_ref.at[:, pl.ds(col * w, w)],
                        out_sems.at[li],
                    ).start()

        for li in range(N_LANES):
            pltpu.make_async_copy(
                comm.at[li, B_SLOT[2]],
                out_ref.at[:, pl.ds(LANES[li][0] * w, w)],
                out_sems.at[li],
            ).wait()

    return pl.pallas_call(
        body,
        out_shape=jax.ShapeDtypeStruct((m_blk, n), jnp.bfloat16),
        in_specs=[
            pl.BlockSpec(memory_space=pl.ANY),
            pl.BlockSpec(memory_space=pl.ANY),
        ],
        out_specs=pl.BlockSpec(memory_space=pl.ANY),
        scratch_shapes=[
            pltpu.VMEM((m, k), jnp.bfloat16),
            pltpu.VMEM((k, n), jnp.bfloat16),
            pltpu.VMEM((2, 256, k), jnp.float32),
            pltpu.VMEM((k, n // N_LANES), jnp.float32),
            pltpu.VMEM((N_LANES, 2, m_blk, n // N_LANES), jnp.bfloat16),
            pltpu.SemaphoreType.DMA((2,)),
            pltpu.SemaphoreType.DMA,
            pltpu.SemaphoreType.DMA((N_LANES, 3)),
            pltpu.SemaphoreType.DMA((N_LANES, 3)),
            pltpu.SemaphoreType.DMA((N_LANES,)),
        ],
        compiler_params=pltpu.CompilerParams(
            collective_id=0,
            vmem_limit_bytes=100 * 1024 * 1024,
        ),
    )(A, B)
